# baseline (speedup 1.0000x reference)
import functools

import jax
import jax.numpy as jnp
from jax import lax
from jax.experimental import pallas as pl
from jax.experimental.pallas import tpu as pltpu
from jax.experimental.pallas import tpu_sc as plsc

VOCAB = 1000000
HIDDEN = 64
BATCH = 16384
SEQ = 16

_NC, _NS = 2, 16
_NW = _NC * _NS
_VB_FULL = VOCAB // 128
_V_TAIL = VOCAB - _VB_FULL * 128
_BPW1 = -(-_VB_FULL // _NW)
_PAIRS = VOCAB // 2
_BB = BATCH // 128
_BB_PER_W = _BB // _NW
_TILES_PER_W = _BB_PER_W * SEQ

_sc_mesh = plsc.VectorSubcoreMesh(core_axis_name="c", subcore_axis_name="s")


def _iota16():
    return lax.broadcasted_iota(jnp.int32, (16,), 0)


def _depad_body(tabT, pairs, buf, obuf, sem_in, sem_out):
    wid = lax.axis_index("s") * _NC + lax.axis_index("c")
    lo = wid * _BPW1
    hi = jnp.minimum(lo + _BPW1, _VB_FULL)
    n = hi - lo
    iota = _iota16()

    def in_copy(vb, sl):
        return pltpu.make_async_copy(
            tabT.at[:, pl.ds(vb * 128, 128)], buf.at[sl], sem_in.at[sl]
        )

    def out_copy(vb, sl):
        return pltpu.make_async_copy(
            obuf.at[sl], pairs.at[pl.ds(vb * 64, 64)], sem_out.at[sl]
        )

    def transpose_block(sl, npairs):
        def jstep(j, carry):
            for k0 in range(0, 128, 16):
                rows = iota + (k0 % 64)
                cols = jnp.zeros((16,), jnp.int32) + (2 * j + k0 // 64)
                val = plsc.load_gather(buf.at[sl], [rows, cols])
                obuf[sl, j, pl.ds(k0, 16)] = val
            return carry

        lax.fori_loop(0, npairs, jstep, 0)

    @pl.when(n > 0)
    def _():
        in_copy(lo, 0).start()

    def step2(ii, carry):
        for sl in range(2):
            i = ii * 2 + sl
            vb = lo + i

            @pl.when(vb < hi)
            def _():
                @pl.when(vb + 1 < hi)
                def _():
                    in_copy(vb + 1, 1 - sl).start()

                in_copy(vb, sl).wait()

                @pl.when(i >= 2)
                def _():
                    out_copy(vb, sl).wait()

                transpose_block(sl, 64)
                out_copy(vb, sl).start()

        return carry

    lax.fori_loop(0, (_BPW1 + 1) // 2, step2, 0)

    @pl.when(n > 0)
    def _():
        out_copy(lo, 0).wait()

    @pl.when(n > 1)
    def _():
        out_copy(lo, 1).wait()


_depad = pl.kernel(
    _depad_body,
    mesh=_sc_mesh,
    out_type=jax.ShapeDtypeStruct((_PAIRS, 128), jnp.float32),
    scratch_types=[
        pltpu.VMEM((2, 64, 128), jnp.float32),
        pltpu.VMEM((2, 64, 128), jnp.float32),
        pltpu.SemaphoreType.DMA((2,)),
        pltpu.SemaphoreType.DMA((2,)),
    ],
    compiler_params=pltpu.CompilerParams(needs_layout_passes=False),
)


_TAIL_PAIR0 = _VB_FULL * 64


def _gather_body(
    pairs, tail_pairs, idsT, out3, ids_v, tail_v, pidx, hb64, rows, obuf,
    sem_g, sem_out,
):
    wid = lax.axis_index("s") * _NC + lax.axis_index("c")
    iota = _iota16()

    pltpu.sync_copy(idsT.at[:, pl.ds(wid * 512, _BB_PER_W * 128)], ids_v)
    pltpu.sync_copy(tail_pairs, tail_v)

    def prep_idx(t, sl):
        s = t // _BB_PER_W
        c = t % _BB_PER_W

        def kstep(kk, carry):
            k0 = kk * 16
            ids = ids_v[s, pl.ds(c * 128 + k0, 16)]
            pidx[sl, pl.ds(k0, 16)] = lax.shift_right_logical(ids, 1)
            hb64[sl, pl.ds(k0, 16)] = lax.shift_left(
                lax.bitwise_and(ids, 1), 6
            )
            return carry

        lax.fori_loop(0, 8, kstep, 0)

    def gather_copy(sl):
        return pltpu.make_async_copy(
            pairs.at[pidx.at[sl]], rows.at[sl], sem_g.at[sl]
        )

    def out_copy(t, sl):
        s = t // _BB_PER_W
        bb = wid * _BB_PER_W + t % _BB_PER_W
        return pltpu.make_async_copy(
            obuf.at[sl], out3.at[s, :, pl.ds(bb * 128, 128)], sem_out.at[sl]
        )

    def extract(sl):
        has_tail = jnp.int32(0)
        for lb0 in range(0, 128, 16):
            pv = pidx[sl, pl.ds(lb0, 16)]
            has_tail = jnp.maximum(has_tail, lax.reduce_max(pv, (0,)))

        @pl.when(has_tail < _TAIL_PAIR0)
        def _():
            def hstep(h, carry):
                for lb0 in range(0, 128, 16):
                    hbv = hb64[sl, pl.ds(lb0, 16)]
                    val = plsc.load_gather(rows.at[sl], [iota + lb0, hbv + h])
                    obuf[sl, h, pl.ds(lb0, 16)] = val
                return carry

            lax.fori_loop(0, HIDDEN, hstep, 0)

        @pl.when(has_tail >= _TAIL_PAIR0)
        def _():
            def hstep(h, carry):
                for lb0 in range(0, 128, 16):
                    hbv = hb64[sl, pl.ds(lb0, 16)]
                    pv = pidx[sl, pl.ds(lb0, 16)]
                    is_tail = pv >= _TAIL_PAIR0
                    val = plsc.load_gather(rows.at[sl], [iota + lb0, hbv + h])
                    tv = plsc.load_gather(
                        tail_v,
                        [jnp.maximum(pv - _TAIL_PAIR0, 0), hbv + h],
                    )
                    obuf[sl, h, pl.ds(lb0, 16)] = jnp.where(is_tail, tv, val)
                return carry

            lax.fori_loop(0, HIDDEN, hstep, 0)

    prep_idx(0, 0)
    gather_copy(0).start()

    def step2(ii, carry):
        for sl in range(2):
            t = ii * 2 + sl
            gather_copy(sl).wait()

            @pl.when(t + 1 < _TILES_PER_W)
            def _():
                prep_idx(t + 1, 1 - sl)
                gather_copy(1 - sl).start()

            @pl.when(t >= 2)
            def _():
                out_copy(t, sl).wait()

            extract(sl)
            out_copy(t, sl).start()
        return carry

    lax.fori_loop(0, _TILES_PER_W // 2, step2, 0)
    out_copy(0, 0).wait()
    out_copy(1, 1).wait()


_gather = pl.kernel(
    _gather_body,
    mesh=_sc_mesh,
    out_type=jax.ShapeDtypeStruct((SEQ, HIDDEN, BATCH), jnp.float32),
    scratch_types=[
        pltpu.VMEM((SEQ, _BB_PER_W * 128), jnp.int32),
        pltpu.VMEM((_V_TAIL // 2, 128), jnp.float32),
        pltpu.VMEM((2, 128), jnp.int32),
        pltpu.VMEM((2, 128), jnp.int32),
        pltpu.VMEM((2, 128, 128), jnp.float32),
        pltpu.VMEM((2, HIDDEN, 128), jnp.float32),
        pltpu.SemaphoreType.DMA((2,)),
        pltpu.SemaphoreType.DMA((2,)),
    ],
    compiler_params=pltpu.CompilerParams(needs_layout_passes=False),
)


def _mask_body(am_ref, out_ref):
    out_ref[...] = am_ref[...] == 0


def kernel(input_ids, attention_mask, table):
    tabT = table.T
    idsT = input_ids.astype(jnp.int32).T
    tail_pairs = table[_VB_FULL * 128 :].reshape(_V_TAIL // 2, 128)
    pairs = _depad(tabT)
    out3 = _gather(pairs, tail_pairs, idsT)
    emb = out3.transpose(2, 0, 1)
    mask = pl.pallas_call(
        _mask_body,
        out_shape=jax.ShapeDtypeStruct((BATCH // 128, 128 * SEQ), jnp.bool_),
    )(attention_mask.reshape(BATCH // 128, 128 * SEQ))
    return emb, mask.reshape(BATCH, SEQ)
